# baseline (speedup 1.0000x reference)
import jax
import jax.numpy as jnp
from jax.experimental import pallas as pl
from jax.experimental.pallas import tpu as pltpu

_FEAT = 8
_POS_DIM = 248
_ROW = _FEAT + _POS_DIM
_MAX_ROWS = 6
_NTAB = 7
_BLKB = 512


def _tile_rows(x, n_target):
    while x.shape[0] < n_target:
        take = min(x.shape[0], n_target - x.shape[0])
        x = jnp.concatenate([x, x[:take]], axis=0)
    return x


def _encode_body(aid_ref, mid_ref, len_ref,
                 data_ref, tab_ref, wnum_ref, wpos_ref,
                 out_un_ref, out_m_ref):
    n_un = aid_ref.shape[0]
    rblk = data_ref.shape[0]
    blkb = rblk // n_un
    n_m = out_m_ref.shape[0] // blkb
    n_pos = wpos_ref.shape[0]

    pos_rows, maxidx, isnum = [], [], []
    embpat = [[] for _ in range(_MAX_ROWS)]
    for c in range(n_un):
        aid = aid_ref[c]
        bid = jnp.minimum(aid, _NTAB)
        tid = jnp.minimum(bid, _NTAB - 1)
        pos_row = wpos_ref[pl.ds(jnp.clip(aid, 0, n_pos - 1), 1), :]
        pos_rows.append(
            jnp.concatenate([jnp.zeros((1, _FEAT), jnp.float32), pos_row],
                            axis=1))
        tbl = tab_ref[pl.ds(tid, 1)]
        numflag = bid == _NTAB
        for k in range(_MAX_ROWS):
            embpat[k].append(
                jnp.where(numflag, jnp.zeros((1, _FEAT), jnp.float32),
                          tbl[0, k:k + 1, :]))
        maxidx.append(jnp.reshape(len_ref[tid] - 1, (1, 1)))
        isnum.append(jnp.reshape(
            jnp.where(numflag, 1.0, 0.0).astype(jnp.float32), (1, 1)))

    pos_t = _tile_rows(jnp.concatenate(pos_rows, axis=0), rblk)
    maxidx_t = _tile_rows(jnp.concatenate(maxidx, axis=0), rblk)
    isnum_t = _tile_rows(jnp.concatenate(isnum, axis=0), rblk)
    emb_t = [_tile_rows(jnp.concatenate(embpat[k], axis=0), rblk)
             for k in range(_MAX_ROWS)]

    dflat = data_ref[...]
    dint = jnp.clip(dflat.astype(jnp.int32), 0, maxidx_t)
    cat8 = jnp.zeros((rblk, _FEAT), jnp.float32)
    for k in range(_MAX_ROWS):
        cat8 = cat8 + jnp.where(dint == k, 1.0, 0.0) * emb_t[k]
    emb8 = cat8 + (isnum_t * dflat) * wnum_ref[0:1, :]
    out_un_ref[...] = pos_t + jnp.pad(emb8, ((0, 0), (0, _POS_DIM)))

    m_rows = []
    for c in range(n_m):
        mid = mid_ref[c]
        bid = jnp.minimum(mid, _NTAB - 1)
        tbl = tab_ref[pl.ds(bid, 1)]
        mrow = len_ref[bid] - 1
        vec8 = jnp.zeros((1, _FEAT), jnp.float32)
        for k in range(_MAX_ROWS):
            vec8 = vec8 + jnp.where(mrow == k, 1.0, 0.0) * tbl[0, k:k + 1, :]
        pos_row = wpos_ref[pl.ds(jnp.clip(mid, 0, n_pos - 1), 1), :]
        m_rows.append(jnp.concatenate([vec8, pos_row], axis=1))
    out_m_ref[...] = _tile_rows(jnp.concatenate(m_rows, axis=0), blkb * n_m)


def kernel(unmasked_data, unmasked_idx, masked_idx, W_Gender, W_Department,
           W_Grade, W_Extracurricular_Activities, W_Internet_Access_at_Home,
           W_Parent_Education_Level, W_Family_Income_Level, W_num, W_pos):
    tables = [W_Gender, W_Department, W_Grade, W_Extracurricular_Activities,
              W_Internet_Access_at_Home, W_Parent_Education_Level,
              W_Family_Income_Level]
    bsz, n_un = unmasked_data.shape
    n_m = masked_idx.shape[1]
    stacked = jnp.stack(
        [jnp.pad(t, ((0, _MAX_ROWS - t.shape[0]), (0, 0))) for t in tables])
    lens = jnp.array([t.shape[0] for t in tables], jnp.int32)
    aid = unmasked_idx[0, :]
    mid = masked_idx[0, :]

    grid = (bsz // _BLKB,)
    out_shapes = (
        jax.ShapeDtypeStruct((bsz * n_un, _ROW), jnp.float32),
        jax.ShapeDtypeStruct((bsz * n_m, _ROW), jnp.float32),
    )
    out_un, out_m = pl.pallas_call(
        _encode_body,
        grid=grid,
        in_specs=[
            pl.BlockSpec(memory_space=pltpu.SMEM),
            pl.BlockSpec(memory_space=pltpu.SMEM),
            pl.BlockSpec(memory_space=pltpu.SMEM),
            pl.BlockSpec((_BLKB * n_un, 1), lambda i: (i, 0)),
            pl.BlockSpec((_NTAB, _MAX_ROWS, _FEAT), lambda i: (0, 0, 0)),
            pl.BlockSpec((1, _FEAT), lambda i: (0, 0)),
            pl.BlockSpec(W_pos.shape, lambda i: (0, 0)),
        ],
        out_specs=[
            pl.BlockSpec((_BLKB * n_un, _ROW), lambda i: (i, 0)),
            pl.BlockSpec((_BLKB * n_m, _ROW), lambda i: (i, 0)),
        ],
        out_shape=out_shapes,
        compiler_params=pltpu.CompilerParams(
            dimension_semantics=("parallel",)),
    )(aid, mid, lens, unmasked_data.reshape(bsz * n_un, 1), stacked, W_num,
      W_pos)
    return (out_un.reshape(bsz, n_un, _ROW), out_m.reshape(bsz, n_m, _ROW))


# speedup vs baseline: 1.6123x; 1.6123x over previous
import jax
import jax.numpy as jnp
from jax.experimental import pallas as pl
from jax.experimental.pallas import tpu as pltpu

_FEAT = 8
_POS_DIM = 248
_ROW = _FEAT + _POS_DIM
_MAX_ROWS = 6
_NTAB = 7
_BLKB = 512


def _encode_body(aid_ref, mid_ref, len_ref,
                 data_ref, tab_ref, wnum_ref, wpos_ref,
                 out_un_ref, out_m_ref):
    blkb, n_un = data_ref.shape
    n_m = out_m_ref.shape[1]
    n_pos = wpos_ref.shape[0]

    pos_rows, maxidx, isnum = [], [], []
    embpat = [[] for _ in range(_MAX_ROWS)]
    for c in range(n_un):
        aid = aid_ref[c]
        bid = jnp.minimum(aid, _NTAB)
        tid = jnp.minimum(bid, _NTAB - 1)
        pos_row = wpos_ref[pl.ds(jnp.clip(aid, 0, n_pos - 1), 1), :]
        pos_rows.append(
            jnp.concatenate([jnp.zeros((1, _FEAT), jnp.float32), pos_row],
                            axis=1))
        tbl = tab_ref[pl.ds(tid, 1)]
        numflag = bid == _NTAB
        for k in range(_MAX_ROWS):
            embpat[k].append(
                jnp.where(numflag, jnp.zeros((1, _FEAT), jnp.float32),
                          tbl[0, k:k + 1, :]))
        maxidx.append(jnp.reshape(len_ref[tid] - 1, (1, 1)))
        isnum.append(jnp.reshape(
            jnp.where(numflag, 1, 0).astype(jnp.int32), (1, 1)))

    pos_pat = jnp.concatenate(pos_rows, axis=0)
    maxidx_pat = jnp.concatenate(maxidx, axis=0)
    isnum_pat = jnp.concatenate(isnum, axis=0)
    emb_pat = [jnp.concatenate(embpat[k], axis=0)
               for k in range(_MAX_ROWS)]
    wnum8 = jnp.broadcast_to(wnum_ref[0:1, :], (n_un, _FEAT))

    cols = []
    for c in range(n_un):
        dcol = data_ref[:, c:c + 1]
        di = jnp.clip(dcol.astype(jnp.int32), 0, maxidx_pat[c, 0])
        acc = jnp.zeros((blkb, _FEAT), jnp.float32)
        for k in range(_MAX_ROWS):
            acc = jnp.where(di == k, emb_pat[k][c:c + 1, :], acc)
        acc = jnp.where(isnum_pat[c, 0] == 1, dcol * wnum8[0:1, :], acc)
        cols.append(acc)
    emb3t = jnp.stack(cols, axis=0)
    emb3 = jnp.transpose(emb3t, (1, 0, 2))
    pad3 = jnp.pad(emb3, ((0, 0), (0, 0), (0, _POS_DIM)))
    out_un_ref[...] = pad3 + jnp.broadcast_to(
        pos_pat[None], (blkb, n_un, _ROW))

    m_rows = []
    for c in range(n_m):
        mid = mid_ref[c]
        bid = jnp.minimum(mid, _NTAB - 1)
        tbl = tab_ref[pl.ds(bid, 1)]
        mrow = len_ref[bid] - 1
        vec8 = jnp.zeros((1, _FEAT), jnp.float32)
        for k in range(_MAX_ROWS):
            vec8 = vec8 + jnp.where(mrow == k, 1.0, 0.0) * tbl[0, k:k + 1, :]
        pos_row = wpos_ref[pl.ds(jnp.clip(mid, 0, n_pos - 1), 1), :]
        m_rows.append(jnp.concatenate([vec8, pos_row], axis=1))
    m_pat = jnp.concatenate(m_rows, axis=0)
    out_m_ref[...] = jnp.broadcast_to(m_pat[None], (blkb, n_m, _ROW))


def kernel(unmasked_data, unmasked_idx, masked_idx, W_Gender, W_Department,
           W_Grade, W_Extracurricular_Activities, W_Internet_Access_at_Home,
           W_Parent_Education_Level, W_Family_Income_Level, W_num, W_pos):
    tables = [W_Gender, W_Department, W_Grade, W_Extracurricular_Activities,
              W_Internet_Access_at_Home, W_Parent_Education_Level,
              W_Family_Income_Level]
    bsz, n_un = unmasked_data.shape
    n_m = masked_idx.shape[1]
    stacked = jnp.stack(
        [jnp.pad(t, ((0, _MAX_ROWS - t.shape[0]), (0, 0))) for t in tables])
    lens = jnp.array([t.shape[0] for t in tables], jnp.int32)
    aid = unmasked_idx[0, :]
    mid = masked_idx[0, :]

    grid = (bsz // _BLKB,)
    out_shapes = (
        jax.ShapeDtypeStruct((bsz, n_un, _ROW), jnp.float32),
        jax.ShapeDtypeStruct((bsz, n_m, _ROW), jnp.float32),
    )
    out_un, out_m = pl.pallas_call(
        _encode_body,
        grid=grid,
        in_specs=[
            pl.BlockSpec(memory_space=pltpu.SMEM),
            pl.BlockSpec(memory_space=pltpu.SMEM),
            pl.BlockSpec(memory_space=pltpu.SMEM),
            pl.BlockSpec((_BLKB, n_un), lambda i: (i, 0)),
            pl.BlockSpec((_NTAB, _MAX_ROWS, _FEAT), lambda i: (0, 0, 0)),
            pl.BlockSpec((1, _FEAT), lambda i: (0, 0)),
            pl.BlockSpec(W_pos.shape, lambda i: (0, 0)),
        ],
        out_specs=[
            pl.BlockSpec((_BLKB, n_un, _ROW), lambda i: (i, 0, 0)),
            pl.BlockSpec((_BLKB, n_m, _ROW), lambda i: (i, 0, 0)),
        ],
        out_shape=out_shapes,
        compiler_params=pltpu.CompilerParams(
            dimension_semantics=("parallel",)),
    )(aid, mid, lens, unmasked_data, stacked, W_num, W_pos)
    return out_un, out_m


# speedup vs baseline: 2.0783x; 1.2890x over previous
import jax
import jax.numpy as jnp
from jax.experimental import pallas as pl
from jax.experimental.pallas import tpu as pltpu

_FEAT = 8
_POS_DIM = 248
_ROW = _FEAT + _POS_DIM
_MAX_ROWS = 6
_NTAB = 7
_BLKB = 512


def _encode_body(aid_ref, mid_ref, len_ref,
                 data_ref, tab_ref, wnum_ref, wpos_ref,
                 out_un_ref, out_m_ref,
                 pos_s, emb_s, aux_s, m_s):
    blkb, n_un = data_ref.shape
    n_m = out_m_ref.shape[1]
    n_pos = wpos_ref.shape[0]

    @pl.when(pl.program_id(0) == 0)
    def _build_patterns():
        for c in range(n_un):
            aid = aid_ref[c]
            bid = jnp.minimum(aid, _NTAB)
            tid = jnp.minimum(bid, _NTAB - 1)
            pos_row = wpos_ref[pl.ds(jnp.clip(aid, 0, n_pos - 1), 1), :]
            pos_s[c:c + 1, :] = jnp.concatenate(
                [jnp.zeros((1, _FEAT), jnp.float32), pos_row], axis=1)
            tbl = tab_ref[pl.ds(tid, 1)]
            numflag = bid == _NTAB
            lanes = pl.ds(c * _FEAT, _FEAT)
            for k in range(_MAX_ROWS):
                emb_s[k:k + 1, lanes] = jnp.where(
                    numflag, jnp.zeros((1, _FEAT), jnp.float32),
                    tbl[0, k:k + 1, :])
            bound = jnp.where(numflag, -1, len_ref[tid] - 1)
            nrow1 = jnp.reshape(bound, (1, 1)).astype(jnp.float32)
            aux_s[0:1, lanes] = jnp.broadcast_to(nrow1, (1, _FEAT))
            aux_s[1:2, lanes] = wnum_ref[0:1, :]
        for c in range(n_m):
            mid = mid_ref[c]
            bid = jnp.minimum(mid, _NTAB - 1)
            tbl = tab_ref[pl.ds(bid, 1)]
            mrow = len_ref[bid] - 1
            vec8 = jnp.zeros((1, _FEAT), jnp.float32)
            for k in range(_MAX_ROWS):
                vec8 = vec8 + jnp.where(mrow == k, 1.0, 0.0) * tbl[0, k:k + 1, :]
            pos_row = wpos_ref[pl.ds(jnp.clip(mid, 0, n_pos - 1), 1), :]
            m_s[c:c + 1, :] = jnp.concatenate([vec8, pos_row], axis=1)

    out_un_ref[...] = jnp.broadcast_to(pos_s[...][None], (blkb, n_un, _ROW))
    out_m_ref[...] = jnp.broadcast_to(m_s[...][None], (blkb, n_m, _ROW))

    w = n_un * _FEAT
    d96 = jnp.concatenate(
        [jnp.broadcast_to(data_ref[:, c:c + 1], (blkb, _FEAT))
         for c in range(n_un)], axis=1)
    di96 = jnp.clip(d96, 0.0, aux_s[0:1, :]).astype(jnp.int32)
    acc = d96 * aux_s[1:2, :]
    for k in range(_MAX_ROWS):
        acc = jnp.where(di96 == k, emb_s[k:k + 1, :], acc)
    emb3t = jnp.stack([acc[:, c * _FEAT:(c + 1) * _FEAT]
                       for c in range(n_un)], axis=0)
    emb3 = jnp.transpose(emb3t, (1, 0, 2))
    out_un_ref[:, :, 0:_FEAT] = emb3


def kernel(unmasked_data, unmasked_idx, masked_idx, W_Gender, W_Department,
           W_Grade, W_Extracurricular_Activities, W_Internet_Access_at_Home,
           W_Parent_Education_Level, W_Family_Income_Level, W_num, W_pos):
    tables = [W_Gender, W_Department, W_Grade, W_Extracurricular_Activities,
              W_Internet_Access_at_Home, W_Parent_Education_Level,
              W_Family_Income_Level]
    bsz, n_un = unmasked_data.shape
    n_m = masked_idx.shape[1]
    stacked = jnp.stack(
        [jnp.pad(t, ((0, _MAX_ROWS - t.shape[0]), (0, 0))) for t in tables])
    lens = jnp.array([t.shape[0] for t in tables], jnp.int32)
    aid = unmasked_idx[0, :]
    mid = masked_idx[0, :]

    grid = (bsz // _BLKB,)
    out_shapes = (
        jax.ShapeDtypeStruct((bsz, n_un, _ROW), jnp.float32),
        jax.ShapeDtypeStruct((bsz, n_m, _ROW), jnp.float32),
    )
    out_un, out_m = pl.pallas_call(
        _encode_body,
        grid=grid,
        in_specs=[
            pl.BlockSpec(memory_space=pltpu.SMEM),
            pl.BlockSpec(memory_space=pltpu.SMEM),
            pl.BlockSpec(memory_space=pltpu.SMEM),
            pl.BlockSpec((_BLKB, n_un), lambda i: (i, 0)),
            pl.BlockSpec((_NTAB, _MAX_ROWS, _FEAT), lambda i: (0, 0, 0)),
            pl.BlockSpec((1, _FEAT), lambda i: (0, 0)),
            pl.BlockSpec(W_pos.shape, lambda i: (0, 0)),
        ],
        out_specs=[
            pl.BlockSpec((_BLKB, n_un, _ROW), lambda i: (i, 0, 0)),
            pl.BlockSpec((_BLKB, n_m, _ROW), lambda i: (i, 0, 0)),
        ],
        out_shape=out_shapes,
        scratch_shapes=[
            pltpu.VMEM((12, _ROW), jnp.float32),
            pltpu.VMEM((_MAX_ROWS, 12 * _FEAT), jnp.float32),
            pltpu.VMEM((2, 12 * _FEAT), jnp.float32),
            pltpu.VMEM((6, _ROW), jnp.float32),
        ],
        compiler_params=pltpu.CompilerParams(
            dimension_semantics=("arbitrary",)),
    )(aid, mid, lens, unmasked_data, stacked, W_num, W_pos)
    return out_un, out_m
